# baseline (speedup 1.0000x reference)
import functools

import jax
import jax.numpy as jnp
from jax import lax
from jax.experimental import pallas as pl
from jax.experimental.pallas import tpu as pltpu
from jax.experimental.pallas import tpu_sc as plsc

NC = 2
NS = 16
NW = NC * NS

C = 512
NB = 2


@functools.partial(jax.jit, static_argnames=("n_pairs", "n_h", "n_b"))
def _sc_gather(W, idx, n_pairs, n_h, n_b):
    btot = idx.shape[0]
    d = W.shape[1]
    cb = n_b // C

    mesh = plsc.VectorSubcoreMesh(core_axis_name="c", subcore_axis_name="s")

    @functools.partial(
        pl.kernel,
        out_type=jax.ShapeDtypeStruct((n_h, n_b, d), jnp.float32),
        mesh=mesh,
        scratch_types=[
            pltpu.VMEM((NB, C), jnp.int32),
            pltpu.VMEM((NB, C, d), jnp.float32),
            pltpu.SemaphoreType.DMA((NB,)),
            pltpu.SemaphoreType.DMA((NB,)),
        ],
        compiler_params=pltpu.CompilerParams(use_tc_tiling_on_sc=False),
    )
    def body(table_hbm, idx_hbm, out_hbm, idx_v, rows_v, gsem, osem):
        wid = lax.axis_index("s") * NC + lax.axis_index("c")

        def pair(p, carry):
            c0 = (p * NB) * NW + wid
            chunks = [c0, c0 + NW]
            for b in range(NB):
                c = chunks[b]
                pltpu.sync_copy(idx_hbm.at[pl.ds(c * C, C)], idx_v.at[b])
                pltpu.make_async_copy(
                    table_hbm.at[idx_v.at[b]], rows_v.at[b], gsem.at[b]
                ).start()
            for b in range(NB):
                c = chunks[b]
                pltpu.make_async_copy(
                    table_hbm.at[idx_v.at[b]], rows_v.at[b], gsem.at[b]
                ).wait()
                pltpu.make_async_copy(
                    rows_v.at[b],
                    out_hbm.at[c // cb, pl.ds((c % cb) * C, C)],
                    osem.at[b],
                ).start()
            for b in range(NB):
                c = chunks[b]
                pltpu.make_async_copy(
                    rows_v.at[b],
                    out_hbm.at[c // cb, pl.ds((c % cb) * C, C)],
                    osem.at[b],
                ).wait()
            return carry

        lax.fori_loop(0, n_pairs, pair, 0)

    return body(W, idx)


def kernel(token_ids, W):
    b, h = token_ids.shape
    d = W.shape[1]
    idx = token_ids.T.reshape(-1).astype(jnp.int32)
    btot = idx.shape[0]
    n_pairs = btot // (NW * NB * C)
    out = _sc_gather(W, idx, n_pairs, h, b)
    return out.transpose(1, 0, 2)
